# Initial kernel scaffold; baseline (speedup 1.0000x reference)
import jax
import jax.numpy as jnp
from jax.experimental import pallas as pl


def kernel(indices, table):
    raise NotImplementedError("write your pallas kernel here")


# speedup vs baseline: 4.6154x; 4.6154x over previous
import functools

import jax
import jax.numpy as jnp
from jax import lax
from jax.experimental import pallas as pl
from jax.experimental.pallas import tpu as pltpu
from jax.experimental.pallas import tpu_sc as plsc

IDX_W = 128
STREAMS = 5
CHUNK = IDX_W * STREAMS


@functools.lru_cache(maxsize=None)
def _make_gather(num_workers: int, b_per_w: int, vocab: int, dim: int):
    n_chunks = b_per_w // CHUNK
    kb = b_per_w // IDX_W
    mesh = plsc.VectorSubcoreMesh(core_axis_name="c", subcore_axis_name="s")
    nc = mesh.num_cores

    @functools.partial(
        pl.kernel,
        out_type=jax.ShapeDtypeStruct((num_workers * b_per_w, dim), jnp.float32),
        mesh=mesh,
        scratch_types=[
            pltpu.VMEM((kb, IDX_W), jnp.int32),
            pltpu.VMEM((2, CHUNK, dim), jnp.float32),
            pltpu.SemaphoreType.DMA,
            pltpu.SemaphoreType.DMA,
            pltpu.SemaphoreType.DMA,
        ],
        compiler_params=pltpu.CompilerParams(use_tc_tiling_on_sc=False),
    )
    def gather_kernel(table_hbm, idx_hbm, out_hbm, idx_v, rows_v,
                      gsem0, gsem1, osem):
        wid = lax.axis_index("s") * nc + lax.axis_index("c")
        base = wid * b_per_w
        pltpu.sync_copy(idx_hbm.at[wid], idx_v)

        @pl.loop(0, n_chunks, step=2)
        def _chunks(g):
            descs = []
            gsems = (gsem0, gsem1)
            for b in range(2):
                c = g + b
                for s in range(STREAMS):
                    d = pltpu.async_copy(
                        table_hbm.at[idx_v.at[c * STREAMS + s]],
                        rows_v.at[b, pl.ds(s * IDX_W, IDX_W)],
                        gsems[b],
                    )
                    descs.append(d)
            for b in range(2):
                c = g + b
                for s in range(STREAMS):
                    descs[b * STREAMS + s].wait()
                pltpu.async_copy(
                    rows_v.at[b],
                    out_hbm.at[pl.ds(base + c * CHUNK, CHUNK)],
                    osem,
                ).wait()

    return gather_kernel


def kernel(indices, table):
    batch, hist = indices.shape
    vocab, dim = table.shape
    total = batch * hist
    info = plsc.get_sparse_core_info()
    nw = info.num_cores * info.num_subcores
    b_per_w = total // nw
    idx3 = indices.reshape(nw, b_per_w // IDX_W, IDX_W)
    out = _make_gather(nw, b_per_w, vocab, dim)(table, idx3)
    return out.reshape(batch, hist, dim)
